# baseline (speedup 1.0000x reference)
import functools

import jax
import jax.numpy as jnp
from jax import lax
from jax.experimental import pallas as pl
from jax.experimental.pallas import tpu as pltpu
from jax.experimental.pallas import tpu_sc as plsc

N = 10000
E = 160000
D_IN = 256
D_EXT = 384
D_H = 512
D_OUT = 40

NW = 32
G = 64
E_W = ((E // NW + G - 1) // G) * G
NCH = E_W // G


_BLK = 1000
_GRID = N // _BLK

_dot = functools.partial(jnp.dot, precision=lax.Precision.HIGHEST,
                         preferred_element_type=jnp.float32)


def _ext_body(x_ref, o_ref):
    o_ref[:, :D_IN] = x_ref[...]
    lane = lax.broadcasted_iota(jnp.int32, (_BLK, D_EXT - D_IN), 1)
    o_ref[:, D_IN:] = jnp.where(lane == 0, 1.0, 0.0).astype(jnp.float32)


def _build_ext(x):
    return pl.pallas_call(
        _ext_body,
        grid=(_GRID,),
        in_specs=[pl.BlockSpec((_BLK, D_IN), lambda i: (i, 0))],
        out_specs=pl.BlockSpec((_BLK, D_EXT), lambda i: (i, 0)),
        out_shape=jax.ShapeDtypeStruct((N, D_EXT), jnp.float32),
    )(x)


def _norm_rows(o):
    nrm = jnp.sqrt(jnp.sum(o * o, axis=1, keepdims=True))
    return o / jnp.maximum(nrm, 1e-12)


def _tc1_body(agg_ref, x_ref, Wl_ref, Wr_ref, b_ref, h_ref, r_ref):
    deg = agg_ref[:, D_IN:D_IN + 1]
    r = 1.0 / jnp.maximum(deg, 1.0)
    o = (_dot(agg_ref[:, :D_IN] * r, Wl_ref[...])
         + _dot(x_ref[...], Wr_ref[...]) + b_ref[...])
    h_ref[...] = jnp.maximum(_norm_rows(o), 0.0)
    r_ref[...] = r


def _tc1(agg_ext, x, Wl, Wr, b):
    return pl.pallas_call(
        _tc1_body,
        grid=(_GRID,),
        in_specs=[
            pl.BlockSpec((_BLK, D_EXT), lambda i: (i, 0)),
            pl.BlockSpec((_BLK, D_IN), lambda i: (i, 0)),
            pl.BlockSpec((D_IN, D_H), lambda i: (0, 0)),
            pl.BlockSpec((D_IN, D_H), lambda i: (0, 0)),
            pl.BlockSpec((1, D_H), lambda i: (0, 0)),
        ],
        out_specs=[
            pl.BlockSpec((_BLK, D_H), lambda i: (i, 0)),
            pl.BlockSpec((_BLK, 1), lambda i: (i, 0)),
        ],
        out_shape=[
            jax.ShapeDtypeStruct((N, D_H), jnp.float32),
            jax.ShapeDtypeStruct((N, 1), jnp.float32),
        ],
    )(agg_ext, x, Wl, Wr, b.reshape(1, D_H))


def _tc2_body(agg_ref, h_ref, r_ref, Wl_ref, Wr_ref, b_ref, o_ref):
    o = (_dot(agg_ref[...] * r_ref[...], Wl_ref[...])
         + _dot(h_ref[...], Wr_ref[...]) + b_ref[...])
    o_ref[...] = jnp.maximum(_norm_rows(o), 0.0)


def _tc2(agg, h, r, Wl, Wr, b):
    return pl.pallas_call(
        _tc2_body,
        grid=(_GRID,),
        in_specs=[
            pl.BlockSpec((_BLK, D_H), lambda i: (i, 0)),
            pl.BlockSpec((_BLK, D_H), lambda i: (i, 0)),
            pl.BlockSpec((_BLK, 1), lambda i: (i, 0)),
            pl.BlockSpec((D_H, D_H), lambda i: (0, 0)),
            pl.BlockSpec((D_H, D_H), lambda i: (0, 0)),
            pl.BlockSpec((1, D_H), lambda i: (0, 0)),
        ],
        out_specs=pl.BlockSpec((_BLK, D_H), lambda i: (i, 0)),
        out_shape=jax.ShapeDtypeStruct((N, D_H), jnp.float32),
    )(agg, h, r, Wl, Wr, b.reshape(1, D_H))


def _tc3_body(agg_ref, h_ref, r_ref, Wl_ref, Wr_ref, b_ref, Wfc_ref, bfc_ref,
              o_ref):
    o = (_dot(agg_ref[...] * r_ref[...], Wl_ref[...])
         + _dot(h_ref[...], Wr_ref[...]) + b_ref[...])
    h3 = _norm_rows(o)
    logits = _dot(h3, Wfc_ref[...]) + bfc_ref[...]
    m = jnp.max(logits, axis=1, keepdims=True)
    e = jnp.exp(logits - m)
    o_ref[...] = e / jnp.sum(e, axis=1, keepdims=True)


def _tc3(agg, h, r, Wl, Wr, b, Wfc, bfc):
    return pl.pallas_call(
        _tc3_body,
        grid=(_GRID,),
        in_specs=[
            pl.BlockSpec((_BLK, D_H), lambda i: (i, 0)),
            pl.BlockSpec((_BLK, D_H), lambda i: (i, 0)),
            pl.BlockSpec((_BLK, 1), lambda i: (i, 0)),
            pl.BlockSpec((D_H, D_H), lambda i: (0, 0)),
            pl.BlockSpec((D_H, D_H), lambda i: (0, 0)),
            pl.BlockSpec((1, D_H), lambda i: (0, 0)),
            pl.BlockSpec((D_H, D_OUT), lambda i: (0, 0)),
            pl.BlockSpec((1, D_OUT), lambda i: (0, 0)),
        ],
        out_specs=pl.BlockSpec((_BLK, D_OUT), lambda i: (i, 0)),
        out_shape=jax.ShapeDtypeStruct((N, D_OUT), jnp.float32),
    )(agg, h, r, Wl, Wr, b.reshape(1, D_H), Wfc, bfc.reshape(1, D_OUT))


@functools.cache
def _make_gather(D: int):
    mesh = plsc.VectorSubcoreMesh(core_axis_name="c", subcore_axis_name="s")

    @functools.partial(
        pl.kernel,
        mesh=mesh,
        out_type=jax.ShapeDtypeStruct((NW * E_W, D), jnp.float32),
        scratch_types=[
            pltpu.VMEM((E_W,), jnp.int32),
            pltpu.VMEM((G,), jnp.int32),
            pltpu.VMEM((G,), jnp.int32),
            pltpu.VMEM((G, D), jnp.float32),
            pltpu.VMEM((G, D), jnp.float32),
            pltpu.SemaphoreType.DMA,
            pltpu.SemaphoreType.DMA,
        ],
    )
    def gather_kernel(table_hbm, src_hbm, out_hbm, swin, gidx0, gidx1,
                      rowbuf0, rowbuf1, sem0, sem1):
        c = lax.axis_index("c")
        s = lax.axis_index("s")
        w = s * 2 + c
        base = pl.multiple_of(w * E_W, 8)
        pltpu.sync_copy(src_hbm.at[pl.ds(base, E_W)], swin)

        gidx = (gidx0, gidx1)
        rowbuf = (rowbuf0, rowbuf1)
        sem = (sem0, sem1)

        def start(k):
            b = k % 2
            for j in range(G // 16):
                gidx[b][pl.ds(j * 16, 16)] = swin[pl.ds(k * G + j * 16, 16)]
            return pltpu.async_copy(table_hbm.at[gidx[b]], rowbuf[b], sem[b])

        pending = start(0)
        for k in range(NCH):
            pending.wait()
            if k + 1 < NCH:
                pending = start(k + 1)
            pltpu.sync_copy(
                rowbuf[k % 2],
                out_hbm.at[pl.ds(pl.multiple_of(w * E_W + k * G, 8), G)])

    return gather_kernel


def _segsum(table, src, dst, D):
    msgs = _make_gather(D)(table, src)
    return jax.ops.segment_sum(msgs, dst, num_segments=N + 8)


def kernel(x, edge_index, W1l, W1r, b1, W2l, W2r, b2, W3l, W3r, b3, Wfc, bfc):
    pad = E_W - E // NW
    src = edge_index[0].astype(jnp.int32).reshape(NW, E // NW)
    dst = edge_index[1].astype(jnp.int32).reshape(NW, E // NW)
    src = jnp.concatenate(
        [src, jnp.zeros((NW, pad), jnp.int32)], axis=1).reshape(-1)
    dst = jnp.concatenate(
        [dst, jnp.full((NW, pad), N, jnp.int32)], axis=1).reshape(-1)

    x_ext = _build_ext(x)
    agg1 = _segsum(x_ext, src, dst, D_EXT)
    h1, r = _tc1(agg1, x, W1l, W1r, b1)
    agg2 = _segsum(h1, src, dst, D_H)
    h2 = _tc2(agg2, h1, r, W2l, W2r, b2)
    agg3 = _segsum(h2, src, dst, D_H)
    return _tc3(agg3, h2, r, W3l, W3r, b3, Wfc, bfc)
